# Initial kernel scaffold; baseline (speedup 1.0000x reference)
import jax
import jax.numpy as jnp
from jax.experimental import pallas as pl


def kernel(node_features, edge_features, from_idx, to_idx, W_node_enc, W_edge_enc, W_msg, W_upd, W_t1, b_t1, W_t2, b_t2, w_scores):
    raise NotImplementedError("write your pallas kernel here")


# speedup vs baseline: 2.7046x; 2.7046x over previous
import jax
import jax.numpy as jnp
from jax import lax
from jax.experimental import pallas as pl
from jax.experimental.pallas import tpu as pltpu
from jax.experimental.pallas import tpu_sc as plsc

_N = 10000
_E = 320000
_D = 128
_DE = 16
_L = 3
_B = 100
_GSZ = 50
_MSS = 64
_TD = 16

_NC = 2
_NS = 16
_NW = _NC * _NS
_EPW = _E // _NW
_K = 80
_NBLK = _EPW // _K
_NP = 10240
_RPS = _NP // _NS
_ZR = 64

_BN = 1000
_BE = 4000


def _encode_body(x_ref, wenc_ref, wmsg_ref, nf_ref, a_ref, b_ref):
    nf = jnp.dot(x_ref[...], wenc_ref[...], preferred_element_type=jnp.float32)
    nf_ref[...] = nf
    a_ref[...] = jnp.dot(nf, wmsg_ref[0:_D], preferred_element_type=jnp.float32)
    b_ref[...] = jnp.dot(nf, wmsg_ref[_D:2 * _D], preferred_element_type=jnp.float32)


def _encode(x, wenc, wmsg):
    return pl.pallas_call(
        _encode_body,
        grid=(_N // _BN,),
        in_specs=[pl.BlockSpec((_BN, _D), lambda i: (i, 0)),
                  pl.BlockSpec((_D, _D), lambda i: (0, 0)),
                  pl.BlockSpec((3 * _D, _D), lambda i: (0, 0))],
        out_specs=[pl.BlockSpec((_BN, _D), lambda i: (i, 0))] * 3,
        out_shape=[jax.ShapeDtypeStruct((_N, _D), jnp.float32)] * 3,
    )(x, wenc, wmsg)


def _efw_body(ef_ref, wee_ref, wmsg_ref, out_ref):
    wc = jnp.dot(wee_ref[...], wmsg_ref[2 * _D:3 * _D],
                 preferred_element_type=jnp.float32)
    out_ref[...] = jnp.dot(ef_ref[...], wc, preferred_element_type=jnp.float32)


def _efw(ef, wee, wmsg):
    return pl.pallas_call(
        _efw_body,
        grid=(_E // _BE,),
        in_specs=[pl.BlockSpec((_BE, _DE), lambda i: (i, 0)),
                  pl.BlockSpec((_DE, _D), lambda i: (0, 0)),
                  pl.BlockSpec((3 * _D, _D), lambda i: (0, 0))],
        out_specs=pl.BlockSpec((_BE, _D), lambda i: (i, 0)),
        out_shape=jax.ShapeDtypeStruct((_E, _D), jnp.float32),
    )(ef, wee, wmsg)


def _edge_body(a_hbm, b_hbm, efw_hbm, fi_hbm, ti_hbm, out_hbm,
               fidx, tidx, ra, rb, re, zb, agg_sh, sa, sb, se):
    cid = lax.axis_index("c")
    sid = lax.axis_index("s")
    wid = cid * _NS + sid

    def zrow(r, carry):
        for k in range(_D // 16):
            zb[r, pl.ds(k * 16, 16)] = jnp.zeros((16,), jnp.float32)
        return carry
    lax.fori_loop(0, _ZR, zrow, 0)
    for i in range(_RPS // _ZR):
        pltpu.sync_copy(zb, agg_sh.at[pl.ds(sid * _RPS + i * _ZR, _ZR), :])
    plsc.subcore_barrier()

    ebase = wid * _EPW

    def blk(j, carry):
        off = pl.multiple_of(ebase + j * _K, 8)
        pltpu.sync_copy(fi_hbm.at[pl.ds(off, _K)], fidx)
        pltpu.sync_copy(ti_hbm.at[pl.ds(off, _K)], tidx)
        ca = pltpu.async_copy(a_hbm.at[fidx], ra, sa)
        cb = pltpu.async_copy(b_hbm.at[tidx], rb, sb)
        ce = pltpu.async_copy(efw_hbm.at[pl.ds(off, _K), :], re, se)
        ca.wait()
        cb.wait()
        ce.wait()

        def row(r, c2):
            for k in range(_D // 16):
                s = pl.ds(k * 16, 16)
                ra[r, s] = jnp.maximum(ra[r, s] + rb[r, s] + re[r, s], 0.0)
            return c2
        lax.fori_loop(0, _K, row, 0)
        pltpu.sync_copy(ra, agg_sh.at[tidx], add=True)
        return carry

    lax.fori_loop(0, _NBLK, blk, 0)
    plsc.subcore_barrier()
    pltpu.sync_copy(agg_sh.at[pl.ds(sid * _RPS, _RPS), :],
                    out_hbm.at[cid, pl.ds(sid * _RPS, _RPS), :])


import functools


@functools.lru_cache(maxsize=None)
def _build_edge_call():
    return pl.kernel(
        _edge_body,
        out_type=jax.ShapeDtypeStruct((_NC, _NP, _D), jnp.float32),
        mesh=plsc.VectorSubcoreMesh(core_axis_name="c", subcore_axis_name="s",
                                    num_cores=_NC, num_subcores=_NS),
        scratch_types=[
        pltpu.VMEM((_K,), jnp.int32),
        pltpu.VMEM((_K,), jnp.int32),
        pltpu.VMEM((_K, _D), jnp.float32),
        pltpu.VMEM((_K, _D), jnp.float32),
        pltpu.VMEM((_K, _D), jnp.float32),
            pltpu.VMEM((_ZR, _D), jnp.float32),
            pltpu.VMEM_SHARED((_NP, _D), jnp.float32),
            pltpu.SemaphoreType.DMA,
            pltpu.SemaphoreType.DMA,
            pltpu.SemaphoreType.DMA,
        ],
    )


def _edge_call(a, b, efw, fi, ti):
    return _build_edge_call()(a, b, efw, fi, ti)


def _update_body(nf_ref, g0_ref, g1_ref, wupd_ref, wmsg_ref,
                 nfo_ref, a_ref, b_ref):
    agg = g0_ref[0] + g1_ref[0]
    h = jnp.maximum(
        jnp.dot(nf_ref[...], wupd_ref[0:_D], preferred_element_type=jnp.float32)
        + jnp.dot(agg, wupd_ref[_D:2 * _D], preferred_element_type=jnp.float32),
        0.0)
    nfo_ref[...] = h
    a_ref[...] = jnp.dot(h, wmsg_ref[0:_D], preferred_element_type=jnp.float32)
    b_ref[...] = jnp.dot(h, wmsg_ref[_D:2 * _D], preferred_element_type=jnp.float32)


def _update(nf, aggp, wupd, wmsg):
    return pl.pallas_call(
        _update_body,
        grid=(_N // _BN,),
        in_specs=[pl.BlockSpec((_BN, _D), lambda i: (i, 0)),
                  pl.BlockSpec((1, _BN, _D), lambda i: (0, i, 0)),
                  pl.BlockSpec((1, _BN, _D), lambda i: (1, i, 0)),
                  pl.BlockSpec((2 * _D, _D), lambda i: (0, 0)),
                  pl.BlockSpec((3 * _D, _D), lambda i: (0, 0))],
        out_specs=[pl.BlockSpec((_BN, _D), lambda i: (i, 0))] * 3,
        out_shape=[jax.ShapeDtypeStruct((_N, _D), jnp.float32)] * 3,
    )(nf, aggp, aggp, wupd, wmsg)


def _tail_body(w_ref, lf_ref, wt1_ref, bt1_ref, wt2_ref, bt2_ref, out_ref):
    b1 = bt1_ref[...]
    b2 = bt2_ref[...]
    row = lax.broadcasted_iota(jnp.int32, (_MSS, 1), 0)
    msk = (row < _GSZ).astype(jnp.float32)
    z = jnp.zeros((_MSS - _GSZ, _D), jnp.float32)
    total = jnp.zeros((), jnp.float32)
    for l in range(_L):
        x = lf_ref[l, 0]
        qp = jnp.concatenate([x[0:_GSZ], z], axis=0)
        cp = jnp.concatenate([x[_GSZ:2 * _GSZ], z], axis=0)
        tq = jnp.maximum(jnp.dot(qp, wt1_ref[...],
                                 preferred_element_type=jnp.float32) + b1, 0.0)
        tc = jnp.maximum(jnp.dot(cp, wt1_ref[...],
                                 preferred_element_type=jnp.float32) + b1, 0.0)
        tq = jnp.dot(tq, wt2_ref[...], preferred_element_type=jnp.float32) + b2
        tc = jnp.dot(tc, wt2_ref[...], preferred_element_type=jnp.float32) + b2
        mq = tq * msk
        mc = tc * msk
        si = lax.dot_general(mq, mc, (((1,), (1,)), ((), ())),
                             preferred_element_type=jnp.float32)
        la = si * 10.0
        for _ in range(10):
            m = jnp.max(la, axis=1, keepdims=True)
            la = la - m - jnp.log(jnp.sum(jnp.exp(la - m), axis=1, keepdims=True))
            m = jnp.max(la, axis=0, keepdims=True)
            la = la - m - jnp.log(jnp.sum(jnp.exp(la - m), axis=0, keepdims=True))
        plan = jnp.exp(la)
        pc = jnp.dot(plan, cp, preferred_element_type=jnp.float32)
        score = jnp.sum(qp - jnp.maximum(qp - pc, 0.0))
        total = total + score * jnp.maximum(w_ref[l], 0.0)
    out_ref[0, 0, 0] = total


def _tail(w_scores, lf, wt1, bt1, wt2, bt2):
    out = pl.pallas_call(
        _tail_body,
        grid=(_B,),
        in_specs=[pl.BlockSpec(memory_space=pltpu.SMEM),
                  pl.BlockSpec((_L, 1, 2 * _GSZ, _D), lambda b: (0, b, 0, 0)),
                  pl.BlockSpec((_D, _TD), lambda b: (0, 0)),
                  pl.BlockSpec((1, _TD), lambda b: (0, 0)),
                  pl.BlockSpec((_TD, _TD), lambda b: (0, 0)),
                  pl.BlockSpec((1, _TD), lambda b: (0, 0))],
        out_specs=pl.BlockSpec((1, 1, 1), lambda b: (b, 0, 0),
                               memory_space=pltpu.SMEM),
        out_shape=jax.ShapeDtypeStruct((_B, 1, 1), jnp.float32),
    )(w_scores, lf, wt1, bt1, wt2, bt2)
    return out[:, 0, 0]


def kernel(node_features, edge_features, from_idx, to_idx, W_node_enc,
           W_edge_enc, W_msg, W_upd, W_t1, b_t1, W_t2, b_t2, w_scores):
    fi = from_idx.astype(jnp.int32)
    ti = to_idx.astype(jnp.int32)
    nf, a, b = _encode(node_features, W_node_enc, W_msg)
    efw = _efw(edge_features, W_edge_enc, W_msg)
    feats = []
    for _ in range(_L):
        aggp = _edge_call(a, b, efw, fi, ti)
        nf, a, b = _update(nf, aggp, W_upd, W_msg)
        feats.append(nf)
    lf = jnp.stack(feats, axis=0).reshape(_L, _B, 2 * _GSZ, _D)
    return _tail(w_scores, lf, W_t1, b_t1.reshape(1, _TD), W_t2,
                 b_t2.reshape(1, _TD))


# speedup vs baseline: 3.2228x; 1.1916x over previous
import jax
import jax.numpy as jnp
from jax import lax
from jax.experimental import pallas as pl
from jax.experimental.pallas import tpu as pltpu
from jax.experimental.pallas import tpu_sc as plsc

_N = 10000
_E = 320000
_D = 128
_DE = 16
_L = 3
_B = 100
_GSZ = 50
_MSS = 64
_TD = 16

_NC = 2
_NS = 16
_NW = _NC * _NS
_EPW = _E // _NW
_K = 48
_NBLK = _EPW // _K
_KT = _EPW - _NBLK * _K
_NP = 10240
_RPS = _NP // _NS
_ZR = 32

_BN = 1000
_BE = 4000


def _dbf(x, w):
    return jnp.dot(x.astype(jnp.bfloat16), w.astype(jnp.bfloat16),
                   preferred_element_type=jnp.float32)

def _encode_body(x_ref, wenc_ref, wmsg_ref, nf_ref, a_ref, b_ref):
    nf = _dbf(x_ref[...], wenc_ref[...])
    nf_ref[...] = nf
    a_ref[...] = _dbf(nf, wmsg_ref[0:_D])
    b_ref[...] = _dbf(nf, wmsg_ref[_D:2 * _D])


def _encode(x, wenc, wmsg):
    return pl.pallas_call(
        _encode_body,
        grid=(_N // _BN,),
        in_specs=[pl.BlockSpec((_BN, _D), lambda i: (i, 0)),
                  pl.BlockSpec((_D, _D), lambda i: (0, 0)),
                  pl.BlockSpec((3 * _D, _D), lambda i: (0, 0))],
        out_specs=[pl.BlockSpec((_BN, _D), lambda i: (i, 0))] * 3,
        out_shape=[jax.ShapeDtypeStruct((_N, _D), jnp.float32)] * 3,
    )(x, wenc, wmsg)


def _efw_body(ef_ref, wee_ref, wmsg_ref, out_ref):
    wc = _dbf(wee_ref[...], wmsg_ref[2 * _D:3 * _D])
    out_ref[...] = _dbf(ef_ref[...], wc)


def _efw(ef, wee, wmsg):
    return pl.pallas_call(
        _efw_body,
        grid=(_E // _BE,),
        in_specs=[pl.BlockSpec((_BE, _DE), lambda i: (i, 0)),
                  pl.BlockSpec((_DE, _D), lambda i: (0, 0)),
                  pl.BlockSpec((3 * _D, _D), lambda i: (0, 0))],
        out_specs=pl.BlockSpec((_BE, _D), lambda i: (i, 0)),
        out_shape=jax.ShapeDtypeStruct((_E, _D), jnp.float32),
    )(ef, wee, wmsg)


def _edge_body(a_hbm, b_hbm, efw_hbm, fi_hbm, ti_hbm, out_hbm,
               fidx0, tidx0, ra0, rb0, re0, fidx1, tidx1, ra1, rb1, re1,
               fidxt, tidxt, rat, rbt, ret,
               zb, agg_sh, sa0, sb0, se0, sa1, sb1, se1):
    cid = lax.axis_index("c")
    sid = lax.axis_index("s")
    wid = cid * _NS + sid
    ebase = wid * _EPW

    bufs = ((fidx0, tidx0, ra0, rb0, re0, sa0, sb0, se0),
            (fidx1, tidx1, ra1, rb1, re1, sa1, sb1, se1))

    def load(j, p):
        fidx, tidx, ra, rb, re, sa, sb, se = bufs[p]
        off = pl.multiple_of(ebase + j * _K, 8)
        pltpu.sync_copy(fi_hbm.at[pl.ds(off, _K)], fidx)
        pltpu.sync_copy(ti_hbm.at[pl.ds(off, _K)], tidx)
        pltpu.async_copy(a_hbm.at[fidx], ra, sa)
        pltpu.async_copy(b_hbm.at[tidx], rb, sb)
        pltpu.async_copy(efw_hbm.at[pl.ds(off, _K), :], re, se)

    def proc(p):
        fidx, tidx, ra, rb, re, sa, sb, se = bufs[p]
        pltpu.make_async_copy(a_hbm.at[fidx], ra, sa).wait()
        pltpu.make_async_copy(b_hbm.at[tidx], rb, sb).wait()
        pltpu.make_async_copy(efw_hbm.at[pl.ds(0, _K), :], re, se).wait()

        def row(r, c2):
            for k in range(_D // 16):
                s = pl.ds(k * 16, 16)
                ra[r, s] = jnp.maximum(ra[r, s] + rb[r, s] + re[r, s], 0.0)
            return c2
        lax.fori_loop(0, _K, row, 0)
        pltpu.sync_copy(ra, agg_sh.at[tidx], add=True)

    def zrow(r, carry):
        for k in range(_D // 16):
            zb[r, pl.ds(k * 16, 16)] = jnp.zeros((16,), jnp.float32)
        return carry
    lax.fori_loop(0, _ZR, zrow, 0)
    for i in range(_RPS // _ZR):
        pltpu.sync_copy(zb, agg_sh.at[pl.ds(sid * _RPS + i * _ZR, _ZR), :])
    plsc.subcore_barrier()

    load(0, 0)
    load(1, 1)

    def body(j2, carry):
        j = j2 * 2

        proc(0)

        @pl.when(j + 2 < _NBLK)
        def _():
            load(j + 2, 0)

        proc(1)

        @pl.when(j + 3 < _NBLK)
        def _():
            load(j + 3, 1)
        return carry

    lax.fori_loop(0, _NBLK // 2, body, 0)

    offt = pl.multiple_of(ebase + _NBLK * _K, 8)
    pltpu.sync_copy(fi_hbm.at[pl.ds(offt, _KT)], fidxt)
    pltpu.sync_copy(ti_hbm.at[pl.ds(offt, _KT)], tidxt)
    ca = pltpu.async_copy(a_hbm.at[fidxt], rat, sa0)
    cb = pltpu.async_copy(b_hbm.at[tidxt], rbt, sb0)
    ce = pltpu.async_copy(efw_hbm.at[pl.ds(offt, _KT), :], ret, se0)
    ca.wait()
    cb.wait()
    ce.wait()

    def trow(r, c2):
        for k in range(_D // 16):
            s = pl.ds(k * 16, 16)
            rat[r, s] = jnp.maximum(rat[r, s] + rbt[r, s] + ret[r, s], 0.0)
        return c2
    lax.fori_loop(0, _KT, trow, 0)
    pltpu.sync_copy(rat, agg_sh.at[tidxt], add=True)
    plsc.subcore_barrier()
    pltpu.sync_copy(agg_sh.at[pl.ds(sid * _RPS, _RPS), :],
                    out_hbm.at[cid, pl.ds(sid * _RPS, _RPS), :])


import functools


@functools.lru_cache(maxsize=None)
def _build_edge_call():
    return pl.kernel(
        _edge_body,
        out_type=jax.ShapeDtypeStruct((_NC, _NP, _D), jnp.float32),
        mesh=plsc.VectorSubcoreMesh(core_axis_name="c", subcore_axis_name="s",
                                    num_cores=_NC, num_subcores=_NS),
        scratch_types=(
            [pltpu.VMEM((_K,), jnp.int32),
             pltpu.VMEM((_K,), jnp.int32),
             pltpu.VMEM((_K, _D), jnp.float32),
             pltpu.VMEM((_K, _D), jnp.float32),
             pltpu.VMEM((_K, _D), jnp.float32)] * 2
            + [pltpu.VMEM((_KT,), jnp.int32),
               pltpu.VMEM((_KT,), jnp.int32),
               pltpu.VMEM((_KT, _D), jnp.float32),
               pltpu.VMEM((_KT, _D), jnp.float32),
               pltpu.VMEM((_KT, _D), jnp.float32)]
            + [pltpu.VMEM((_ZR, _D), jnp.float32),
               pltpu.VMEM_SHARED((_NP, _D), jnp.float32)]
            + [pltpu.SemaphoreType.DMA] * 6
        ),
    )


def _edge_call(a, b, efw, fi, ti):
    return _build_edge_call()(a, b, efw, fi, ti)


def _update_body(nf_ref, g0_ref, g1_ref, wupd_ref, wmsg_ref,
                 nfo_ref, a_ref, b_ref):
    agg = g0_ref[0] + g1_ref[0]
    nfb = nf_ref[...].astype(jnp.bfloat16)
    agb = agg.astype(jnp.bfloat16)
    wu1 = wupd_ref[0:_D].astype(jnp.bfloat16)
    wu2 = wupd_ref[_D:2 * _D].astype(jnp.bfloat16)
    h = jnp.maximum(
        _dbf(nfb, wu1)
        + _dbf(agb, wu2),
        0.0)
    nfo_ref[...] = h
    a_ref[...] = _dbf(h, wmsg_ref[0:_D])
    b_ref[...] = _dbf(h, wmsg_ref[_D:2 * _D])


def _update(nf, aggp, wupd, wmsg):
    return pl.pallas_call(
        _update_body,
        grid=(_N // _BN,),
        in_specs=[pl.BlockSpec((_BN, _D), lambda i: (i, 0)),
                  pl.BlockSpec((1, _BN, _D), lambda i: (0, i, 0)),
                  pl.BlockSpec((1, _BN, _D), lambda i: (1, i, 0)),
                  pl.BlockSpec((2 * _D, _D), lambda i: (0, 0)),
                  pl.BlockSpec((3 * _D, _D), lambda i: (0, 0))],
        out_specs=[pl.BlockSpec((_BN, _D), lambda i: (i, 0))] * 3,
        out_shape=[jax.ShapeDtypeStruct((_N, _D), jnp.float32)] * 3,
    )(nf, aggp, aggp, wupd, wmsg)


def _tail_body(w_ref, lf_ref, wt1_ref, bt1_ref, wt2_ref, bt2_ref, out_ref):
    b1 = bt1_ref[...]
    b2 = bt2_ref[...]
    row = lax.broadcasted_iota(jnp.int32, (_MSS, 1), 0)
    msk = (row < _GSZ).astype(jnp.float32)
    z = jnp.zeros((_MSS - _GSZ, _D), jnp.float32)
    total = jnp.zeros((), jnp.float32)
    for l in range(_L):
        x = lf_ref[l, 0]
        qp = jnp.concatenate([x[0:_GSZ], z], axis=0)
        cp = jnp.concatenate([x[_GSZ:2 * _GSZ], z], axis=0)
        tq = jnp.maximum(_dbf(qp, wt1_ref[...]) + b1, 0.0)
        tc = jnp.maximum(_dbf(cp, wt1_ref[...]) + b1, 0.0)
        tq = _dbf(tq, wt2_ref[...]) + b2
        tc = _dbf(tc, wt2_ref[...]) + b2
        mq = tq * msk
        mc = tc * msk
        si = lax.dot_general(mq, mc, (((1,), (1,)), ((), ())))
        la = si * 10.0
        for _ in range(10):
            m = jnp.max(la, axis=1, keepdims=True)
            la = la - m - jnp.log(jnp.sum(jnp.exp(la - m), axis=1, keepdims=True))
            m = jnp.max(la, axis=0, keepdims=True)
            la = la - m - jnp.log(jnp.sum(jnp.exp(la - m), axis=0, keepdims=True))
        plan = jnp.exp(la)
        pc = _dbf(plan, cp)
        score = jnp.sum(qp - jnp.maximum(qp - pc, 0.0))
        total = total + score * jnp.maximum(w_ref[l], 0.0)
    out_ref[0, 0, 0] = total


def _tail(w_scores, lf, wt1, bt1, wt2, bt2):
    out = pl.pallas_call(
        _tail_body,
        grid=(_B,),
        in_specs=[pl.BlockSpec(memory_space=pltpu.SMEM),
                  pl.BlockSpec((_L, 1, 2 * _GSZ, _D), lambda b: (0, b, 0, 0)),
                  pl.BlockSpec((_D, _TD), lambda b: (0, 0)),
                  pl.BlockSpec((1, _TD), lambda b: (0, 0)),
                  pl.BlockSpec((_TD, _TD), lambda b: (0, 0)),
                  pl.BlockSpec((1, _TD), lambda b: (0, 0))],
        out_specs=pl.BlockSpec((1, 1, 1), lambda b: (b, 0, 0),
                               memory_space=pltpu.SMEM),
        out_shape=jax.ShapeDtypeStruct((_B, 1, 1), jnp.float32),
    )(w_scores, lf, wt1, bt1, wt2, bt2)
    return out[:, 0, 0]


def kernel(node_features, edge_features, from_idx, to_idx, W_node_enc,
           W_edge_enc, W_msg, W_upd, W_t1, b_t1, W_t2, b_t2, w_scores):
    fi = from_idx.astype(jnp.int32)
    ti = to_idx.astype(jnp.int32)
    nf, a, b = _encode(node_features, W_node_enc, W_msg)
    efw = _efw(edge_features, W_edge_enc, W_msg)
    feats = []
    for _ in range(_L):
        aggp = _edge_call(a, b, efw, fi, ti)
        nf, a, b = _update(nf, aggp, W_upd, W_msg)
        feats.append(nf)
    lf = jnp.stack(feats, axis=0).reshape(_L, _B, 2 * _GSZ, _D)
    return _tail(w_scores, lf, W_t1, b_t1.reshape(1, _TD), W_t2,
                 b_t2.reshape(1, _TD))
